# Initial kernel scaffold; baseline (speedup 1.0000x reference)
import jax
import jax.numpy as jnp
from jax.experimental import pallas as pl


def kernel(edge_src, edge_dst, edge_weight_cutoff, edge_attr, node_feat, W1, W2, W3, Ws, Wv):
    raise NotImplementedError("write your pallas kernel here")


# speedup vs baseline: 19.5753x; 19.5753x over previous
import functools
import math

import jax
import jax.numpy as jnp
from jax import lax
from jax.experimental import pallas as pl
from jax.experimental.pallas import tpu as pltpu
from jax.experimental.pallas import tpu_sc as plsc

_N = 50000
_E = 800000
_D = 48
_D0 = 24
_D1 = 16
_NC = 2
_NS = 16
_NW = _NC * _NS
_NPAD = 50176
_RPT = _NPAD // _NS
_EPW = _E // _NW
_EPS = _E // _NS
_GCH = 40
_SCH = 40

@functools.cache
def _make_sc_kernels():
    mesh = plsc.VectorSubcoreMesh(
        core_axis_name="c", subcore_axis_name="s",
        num_cores=_NC, num_subcores=_NS,
    )
    params = pltpu.CompilerParams(use_tc_tiling_on_sc=False)
    gather = functools.partial(
        pl.kernel,
        out_type=jax.ShapeDtypeStruct((_E, _D), jnp.float32),
        mesh=mesh,
        compiler_params=params,
        scratch_types=[
            pltpu.VMEM((_GCH,), jnp.int32),
            pltpu.VMEM((_GCH, _D), jnp.float32),
            pltpu.SemaphoreType.DMA,
        ],
    )(_sc_gather_body)
    scatter = functools.partial(
        pl.kernel,
        out_type=(
            jax.ShapeDtypeStruct((_NPAD, _D0), jnp.float32),
            jax.ShapeDtypeStruct((_NPAD, _D1), jnp.float32),
        ),
        mesh=mesh,
        compiler_params=params,
        scratch_types=[
            pltpu.VMEM((1, _SCH), jnp.int32),
            pltpu.VMEM((_SCH, _D0), jnp.float32),
            pltpu.VMEM((_SCH, _D1), jnp.float32),
            pltpu.VMEM_SHARED((_NPAD, _D0), jnp.float32),
            pltpu.VMEM_SHARED((_NPAD, _D1), jnp.float32),
            pltpu.SemaphoreType.DMA,
        ],
    )(_sc_scatter_body)
    return gather, scatter


def _sc_gather_body(tab, idx, out, idxb, rowb, sem):
    wid = lax.axis_index("s") * _NC + lax.axis_index("c")
    base = wid * _EPW

    def body(j, carry):
        off = base + j * _GCH
        pltpu.sync_copy(idx.at[pl.ds(off, _GCH)], idxb)
        pltpu.async_copy(tab.at[idxb], rowb, sem).wait()
        pltpu.sync_copy(rowb, out.at[pl.ds(off, _GCH)])
        return carry

    lax.fori_loop(0, _EPW // _GCH, body, 0)


def _sc_scatter_body(y0, y1, dst, z0, z1, o0, o1, idxb, yb0, yb1, acc0, acc1, sem):
    c = lax.axis_index("c")
    s = lax.axis_index("s")
    r0 = s * _RPT
    base = s * _EPS

    @pl.when(c == 0)
    def _():
        pltpu.sync_copy(z0.at[pl.ds(r0, _RPT)], acc0.at[pl.ds(r0, _RPT)])

    @pl.when(c == 1)
    def _():
        pltpu.sync_copy(z1.at[pl.ds(r0, _RPT)], acc1.at[pl.ds(r0, _RPT)])

    plsc.subcore_barrier()

    @pl.when(c == 0)
    def _():
        def body(j, carry):
            off = base + j * _SCH
            pltpu.sync_copy(dst.at[pl.ds(off, _SCH)], idxb.at[0])
            pltpu.sync_copy(y0.at[pl.ds(off, _SCH)], yb0)
            pltpu.sync_copy(yb0, acc0.at[idxb.at[0]], add=True)
            return carry

        lax.fori_loop(0, _EPS // _SCH, body, 0)

    @pl.when(c == 1)
    def _():
        def body(j, carry):
            off = base + j * _SCH
            pltpu.sync_copy(dst.at[pl.ds(off, _SCH)], idxb.at[0])
            pltpu.sync_copy(y1.at[pl.ds(off, _SCH)], yb1)
            pltpu.sync_copy(yb1, acc1.at[idxb.at[0]], add=True)
            return carry

        lax.fori_loop(0, _EPS // _SCH, body, 0)

    plsc.subcore_barrier()

    @pl.when(c == 0)
    def _():
        pltpu.sync_copy(acc0.at[pl.ds(r0, _RPT)], o0.at[pl.ds(r0, _RPT)])

    @pl.when(c == 1)
    def _():
        pltpu.sync_copy(acc1.at[pl.ds(r0, _RPT)], o1.at[pl.ds(r0, _RPT)])


_BE = 2000
_INV_SQRT3 = 1.0 / math.sqrt(3.0)


def _tc_body(attr, cut, src, w1, w2, w3, ws, wv, y0, y1):
    inv = attr[:, :16]
    h = jnp.maximum(jnp.dot(inv, w1[:], preferred_element_type=jnp.float32) * 0.25, 0.0)
    h = jnp.maximum(jnp.dot(h, w2[:], preferred_element_type=jnp.float32) * 0.125, 0.0)
    f = jnp.dot(h, w3[:], preferred_element_type=jnp.float32) * 0.125
    fs = f[:, :24]
    fv = f[:, 24:48]
    cw = cut[:]
    s_ = src[:, :16]
    vx = src[:, 16:24]
    vy = src[:, 24:32]
    vz = src[:, 32:40]
    ex = attr[:, 16:17]
    ey = attr[:, 17:18]
    ez = attr[:, 18:19]
    tp0 = (vx * ex + vy * ey + vz * ez) * _INV_SQRT3
    ms = jnp.concatenate([tp0, s_], axis=1) * fs * cw
    ys = jnp.dot(ms, ws[:], preferred_element_type=jnp.float32)
    mx = jnp.concatenate([s_ * ex, vx], axis=1) * fv * cw
    my = jnp.concatenate([s_ * ey, vy], axis=1) * fv * cw
    mz = jnp.concatenate([s_ * ez, vz], axis=1) * fv * cw
    yx = jnp.dot(mx, wv[:], preferred_element_type=jnp.float32)
    yy = jnp.dot(my, wv[:], preferred_element_type=jnp.float32)
    yz = jnp.dot(mz, wv[:], preferred_element_type=jnp.float32)
    y0[...] = jnp.concatenate([ys, yx], axis=1)
    y1[...] = jnp.concatenate([yy, yz], axis=1)


_tc_compute = pl.pallas_call(
    _tc_body,
    grid=(_E // _BE,),
    in_specs=[
        pl.BlockSpec((_BE, 19), lambda i: (i, 0)),
        pl.BlockSpec((_BE, 1), lambda i: (i, 0)),
        pl.BlockSpec((_BE, _D), lambda i: (i, 0)),
        pl.BlockSpec((16, 64), lambda i: (0, 0)),
        pl.BlockSpec((64, 64), lambda i: (0, 0)),
        pl.BlockSpec((64, 48), lambda i: (0, 0)),
        pl.BlockSpec((24, 16), lambda i: (0, 0)),
        pl.BlockSpec((24, 8), lambda i: (0, 0)),
    ],
    out_specs=[
        pl.BlockSpec((_BE, _D0), lambda i: (i, 0)),
        pl.BlockSpec((_BE, _D1), lambda i: (i, 0)),
    ],
    out_shape=[
        jax.ShapeDtypeStruct((_E, _D0), jnp.float32),
        jax.ShapeDtypeStruct((_E, _D1), jnp.float32),
    ],
)


def kernel(edge_src, edge_dst, edge_weight_cutoff, edge_attr, node_feat,
           W1, W2, W3, Ws, Wv):
    s = node_feat[:, :16]
    v = node_feat[:, 16:].reshape(_N, 8, 3)
    node_tab = jnp.concatenate(
        [s, v[:, :, 0], v[:, :, 1], v[:, :, 2],
         jnp.zeros((_N, 8), jnp.float32)], axis=1)
    src_idx = edge_src.astype(jnp.int32)
    dst_idx = edge_dst.astype(jnp.int32)

    sc_gather, sc_scatter = _make_sc_kernels()
    src_feat = sc_gather(node_tab, src_idx)
    y0, y1 = _tc_compute(edge_attr, edge_weight_cutoff.reshape(_E, 1),
                         src_feat, W1, W2, W3, Ws, Wv)
    z0 = jnp.zeros((_NPAD, _D0), jnp.float32)
    z1 = jnp.zeros((_NPAD, _D1), jnp.float32)
    o0, o1 = sc_scatter(y0, y1, dst_idx, z0, z1)

    ys = o0[:_N, :16]
    yv = jnp.stack([o0[:_N, 16:24], o1[:_N, :8], o1[:_N, 8:16]],
                   axis=-1).reshape(_N, 24)
    return jnp.concatenate([ys, yv], axis=1)


# speedup vs baseline: 27.0008x; 1.3793x over previous
import functools
import math

import jax
import jax.numpy as jnp
from jax import lax
from jax.experimental import pallas as pl
from jax.experimental.pallas import tpu as pltpu
from jax.experimental.pallas import tpu_sc as plsc

_N = 50000
_E = 800000
_D = 48
_D0 = 24
_D1 = 16
_NC = 2
_NS = 16
_NW = _NC * _NS
_NPAD = 50176
_RPT = _NPAD // _NS

_L = 128
_EROWS = _E // _L
_GRPT = _EROWS // _NW
_GEXTRA = _EROWS - _GRPT * _NW
_GR = 5
_GNIT = _GRPT // _GR
_SRPT = _EROWS // _NS
_SEXTRA = _EROWS - _SRPT * _NS
_SNIT = _SRPT // _GR


@functools.cache
def _make_sc_kernels():
    mesh = plsc.VectorSubcoreMesh(
        core_axis_name="c", subcore_axis_name="s",
        num_cores=_NC, num_subcores=_NS,
    )
    params = pltpu.CompilerParams(use_tc_tiling_on_sc=False)
    gather = functools.partial(
        pl.kernel,
        out_type=jax.ShapeDtypeStruct((_E, _D), jnp.float32),
        mesh=mesh,
        compiler_params=params,
        scratch_types=[
            pltpu.VMEM((_GRPT + 1, _L), jnp.int32),
            pltpu.VMEM((_GR * _L, _D), jnp.float32),
            pltpu.SemaphoreType.DMA,
        ],
    )(_sc_gather_body)
    scatter = functools.partial(
        pl.kernel,
        out_type=(
            jax.ShapeDtypeStruct((_NPAD, _D0), jnp.float32),
            jax.ShapeDtypeStruct((_NPAD, _D0), jnp.float32),
        ),
        mesh=mesh,
        compiler_params=params,
        scratch_types=[
            pltpu.VMEM((_GR, _L), jnp.int32),
            pltpu.VMEM((_GR * _L, _D0), jnp.float32),
            pltpu.VMEM_SHARED((_NPAD, _D0), jnp.float32),
            pltpu.SemaphoreType.DMA,
        ],
    )(_sc_scatter_body)
    return gather, scatter


def _sc_gather_body(tab, idx2, out, idxb, rowb, sem):
    wid = lax.axis_index("s") * _NC + lax.axis_index("c")
    base_row = wid * _GRPT
    pltpu.sync_copy(idx2.at[pl.ds(base_row, _GRPT)], idxb.at[pl.ds(0, _GRPT)])

    @pl.when(wid < _GEXTRA)
    def _():
        pltpu.sync_copy(idx2.at[pl.ds(_GRPT * _NW + wid, 1)],
                        idxb.at[pl.ds(_GRPT, 1)])

    def body(g, carry):
        row = g * _GR
        copies = [
            pltpu.async_copy(tab.at[idxb.at[row + k]],
                             rowb.at[pl.ds(k * _L, _L)], sem)
            for k in range(_GR)
        ]
        for cp in copies:
            cp.wait()
        pltpu.sync_copy(rowb, out.at[pl.ds((base_row + row) * _L, _GR * _L)])
        return carry

    lax.fori_loop(0, _GNIT, body, 0)

    @pl.when(wid < _GEXTRA)
    def _():
        pltpu.async_copy(tab.at[idxb.at[_GRPT]],
                         rowb.at[pl.ds(0, _L)], sem).wait()
        pltpu.sync_copy(rowb.at[pl.ds(0, _L)],
                        out.at[pl.ds((_GRPT * _NW + wid) * _L, _L)])


def _sc_scatter_body(y0, y1, dst2, z, o0, o1, idxb, yb, acc, sem):
    c = lax.axis_index("c")
    s = lax.axis_index("s")
    r0 = s * _RPT
    base_row = s * _SRPT

    pltpu.sync_copy(z.at[pl.ds(r0, _RPT)], acc.at[pl.ds(r0, _RPT)])
    plsc.subcore_barrier()

    @pl.when(c == 0)
    def _():
        def body(g, carry):
            row = base_row + g * _GR
            pltpu.sync_copy(dst2.at[pl.ds(row, _GR)], idxb)
            pltpu.sync_copy(y0.at[pl.ds(row * _L, _GR * _L)], yb)
            copies = [
                pltpu.async_copy(yb.at[pl.ds(k * _L, _L)],
                                 acc.at[idxb.at[k]], sem, add=True)
                for k in range(_GR)
            ]
            for cp in copies:
                cp.wait()
            return carry

        lax.fori_loop(0, _SNIT, body, 0)

        @pl.when(s < _SEXTRA)
        def _():
            xrow = _SRPT * _NS + s
            pltpu.sync_copy(dst2.at[pl.ds(xrow, 1)], idxb.at[pl.ds(0, 1)])
            pltpu.sync_copy(y0.at[pl.ds(xrow * _L, _L)], yb.at[pl.ds(0, _L)])
            pltpu.async_copy(yb.at[pl.ds(0, _L)],
                             acc.at[idxb.at[0]], sem, add=True).wait()

    @pl.when(c == 1)
    def _():
        def body(g, carry):
            row = base_row + g * _GR
            pltpu.sync_copy(dst2.at[pl.ds(row, _GR)], idxb)
            pltpu.sync_copy(y1.at[pl.ds(row * _L, _GR * _L)], yb)
            copies = [
                pltpu.async_copy(yb.at[pl.ds(k * _L, _L)],
                                 acc.at[idxb.at[k]], sem, add=True)
                for k in range(_GR)
            ]
            for cp in copies:
                cp.wait()
            return carry

        lax.fori_loop(0, _SNIT, body, 0)

        @pl.when(s < _SEXTRA)
        def _():
            xrow = _SRPT * _NS + s
            pltpu.sync_copy(dst2.at[pl.ds(xrow, 1)], idxb.at[pl.ds(0, 1)])
            pltpu.sync_copy(y1.at[pl.ds(xrow * _L, _L)], yb.at[pl.ds(0, _L)])
            pltpu.async_copy(yb.at[pl.ds(0, _L)],
                             acc.at[idxb.at[0]], sem, add=True).wait()

    plsc.subcore_barrier()

    @pl.when(c == 0)
    def _():
        pltpu.sync_copy(acc.at[pl.ds(r0, _RPT)], o0.at[pl.ds(r0, _RPT)])

    @pl.when(c == 1)
    def _():
        pltpu.sync_copy(acc.at[pl.ds(r0, _RPT)], o1.at[pl.ds(r0, _RPT)])


_BE = 2000
_INV_SQRT3 = 1.0 / math.sqrt(3.0)


def _bf(x):
    return x.astype(jnp.bfloat16)


def _tc_body(attr, cut, src, w1, w2, w3, ws, wv, y0, y1):
    inv = attr[:, :16]
    h = jnp.maximum(
        jnp.dot(_bf(inv), _bf(w1[:]), preferred_element_type=jnp.float32) * 0.25,
        0.0)
    h = jnp.maximum(
        jnp.dot(_bf(h), _bf(w2[:]), preferred_element_type=jnp.float32) * 0.125,
        0.0)
    f = jnp.dot(_bf(h), _bf(w3[:]), preferred_element_type=jnp.float32) * 0.125
    fs = f[:, :24]
    fv = f[:, 24:48]
    cw = cut[:]
    s_ = src[:, :16]
    vx = src[:, 16:24]
    vy = src[:, 24:32]
    vz = src[:, 32:40]
    ex = attr[:, 16:17]
    ey = attr[:, 17:18]
    ez = attr[:, 18:19]
    tp0 = (vx * ex + vy * ey + vz * ez) * _INV_SQRT3
    ms = jnp.concatenate([tp0, s_], axis=1) * fs * cw
    ys = jnp.dot(_bf(ms), _bf(ws[:]), preferred_element_type=jnp.float32)
    mx = jnp.concatenate([s_ * ex, vx], axis=1) * fv * cw
    my = jnp.concatenate([s_ * ey, vy], axis=1) * fv * cw
    mz = jnp.concatenate([s_ * ez, vz], axis=1) * fv * cw
    yx = jnp.dot(_bf(mx), _bf(wv[:]), preferred_element_type=jnp.float32)
    yy = jnp.dot(_bf(my), _bf(wv[:]), preferred_element_type=jnp.float32)
    yz = jnp.dot(_bf(mz), _bf(wv[:]), preferred_element_type=jnp.float32)
    y0[...] = jnp.concatenate([ys, yx], axis=1)
    y1[...] = jnp.concatenate([yy, yz, jnp.zeros((yy.shape[0], 8), jnp.float32)],
                              axis=1)


_tc_compute = pl.pallas_call(
    _tc_body,
    grid=(_E // _BE,),
    in_specs=[
        pl.BlockSpec((_BE, 19), lambda i: (i, 0)),
        pl.BlockSpec((_BE, 1), lambda i: (i, 0)),
        pl.BlockSpec((_BE, _D), lambda i: (i, 0)),
        pl.BlockSpec((16, 64), lambda i: (0, 0)),
        pl.BlockSpec((64, 64), lambda i: (0, 0)),
        pl.BlockSpec((64, 48), lambda i: (0, 0)),
        pl.BlockSpec((24, 16), lambda i: (0, 0)),
        pl.BlockSpec((24, 8), lambda i: (0, 0)),
    ],
    out_specs=[
        pl.BlockSpec((_BE, _D0), lambda i: (i, 0)),
        pl.BlockSpec((_BE, _D0), lambda i: (i, 0)),
    ],
    out_shape=[
        jax.ShapeDtypeStruct((_E, _D0), jnp.float32),
        jax.ShapeDtypeStruct((_E, _D0), jnp.float32),
    ],
)


def kernel(edge_src, edge_dst, edge_weight_cutoff, edge_attr, node_feat,
           W1, W2, W3, Ws, Wv):
    s = node_feat[:, :16]
    v = node_feat[:, 16:].reshape(_N, 8, 3)
    node_tab = jnp.concatenate(
        [s, v[:, :, 0], v[:, :, 1], v[:, :, 2],
         jnp.zeros((_N, 8), jnp.float32)], axis=1)
    src_idx = edge_src.astype(jnp.int32).reshape(_EROWS, _L)
    dst_idx = edge_dst.astype(jnp.int32).reshape(_EROWS, _L)

    sc_gather, sc_scatter = _make_sc_kernels()
    src_feat = sc_gather(node_tab, src_idx)
    y0, y1 = _tc_compute(edge_attr, edge_weight_cutoff.reshape(_E, 1),
                         src_feat, W1, W2, W3, Ws, Wv)
    z = jnp.zeros((_NPAD, _D0), jnp.float32)
    o0, o1 = sc_scatter(y0, y1, dst_idx, z)

    ys = o0[:_N, :16]
    yv = jnp.stack([o0[:_N, 16:24], o1[:_N, :8], o1[:_N, 8:16]],
                   axis=-1).reshape(_N, 24)
    return jnp.concatenate([ys, yv], axis=1)


# speedup vs baseline: 39.0095x; 1.4448x over previous
import functools
import math

import jax
import jax.numpy as jnp
from jax import lax
from jax.experimental import pallas as pl
from jax.experimental.pallas import tpu as pltpu
from jax.experimental.pallas import tpu_sc as plsc

_N = 50000
_E = 800000
_D = 48
_D0 = 24
_D1 = 16
_NC = 2
_NS = 16
_NW = _NC * _NS
_NPAD = 50176
_RPT = _NPAD // _NS

_L = 128
_EROWS = _E // _L
_GRPT = _EROWS // _NW
_GEXTRA = _EROWS - _GRPT * _NW
_GR = 5
_GNIT = _GRPT // _GR
_SRPT = _EROWS // _NS
_SEXTRA = _EROWS - _SRPT * _NS
_SNIT = _SRPT // _GR


@functools.cache
def _make_sc_kernels():
    mesh = plsc.VectorSubcoreMesh(
        core_axis_name="c", subcore_axis_name="s",
        num_cores=_NC, num_subcores=_NS,
    )
    params = pltpu.CompilerParams(use_tc_tiling_on_sc=False)
    gather = functools.partial(
        pl.kernel,
        out_type=jax.ShapeDtypeStruct((_E, _D), jnp.float32),
        mesh=mesh,
        compiler_params=params,
        scratch_types=[
            pltpu.VMEM((_GRPT + 1, _L), jnp.int32),
            pltpu.VMEM((_GR * _L, _D), jnp.float32),
            pltpu.SemaphoreType.DMA,
        ],
    )(_sc_gather_body)
    scatter = functools.partial(
        pl.kernel,
        out_type=(
            jax.ShapeDtypeStruct((_NPAD, _D0), jnp.float32),
            jax.ShapeDtypeStruct((_NPAD, _D0), jnp.float32),
        ),
        mesh=mesh,
        compiler_params=params,
        scratch_types=[
            pltpu.VMEM((_GR, _L), jnp.int32),
            pltpu.VMEM((_GR * _L, _D0), jnp.float32),
            pltpu.VMEM_SHARED((_NPAD, _D0), jnp.float32),
            pltpu.SemaphoreType.DMA,
        ],
    )(_sc_scatter_body)
    return gather, scatter


def _sc_gather_body(tab, idx2, out, idxb, rowb, sem):
    wid = lax.axis_index("s") * _NC + lax.axis_index("c")
    base_row = wid * _GRPT
    pltpu.sync_copy(idx2.at[pl.ds(base_row, _GRPT)], idxb.at[pl.ds(0, _GRPT)])

    @pl.when(wid < _GEXTRA)
    def _():
        pltpu.sync_copy(idx2.at[pl.ds(_GRPT * _NW + wid, 1)],
                        idxb.at[pl.ds(_GRPT, 1)])

    def body(g, carry):
        row = g * _GR
        copies = [
            pltpu.async_copy(tab.at[idxb.at[row + k]],
                             rowb.at[pl.ds(k * _L, _L)], sem)
            for k in range(_GR)
        ]
        for cp in copies:
            cp.wait()
        pltpu.sync_copy(rowb, out.at[pl.ds((base_row + row) * _L, _GR * _L)])
        return carry

    lax.fori_loop(0, _GNIT, body, 0)

    @pl.when(wid < _GEXTRA)
    def _():
        pltpu.async_copy(tab.at[idxb.at[_GRPT]],
                         rowb.at[pl.ds(0, _L)], sem).wait()
        pltpu.sync_copy(rowb.at[pl.ds(0, _L)],
                        out.at[pl.ds((_GRPT * _NW + wid) * _L, _L)])


def _sc_scatter_body(y0, y1, dst2, z, o0, o1, idxb, yb, acc, sem):
    c = lax.axis_index("c")
    s = lax.axis_index("s")
    r0 = s * _RPT
    base_row = s * _SRPT

    pltpu.sync_copy(z.at[pl.ds(r0, _RPT)], acc.at[pl.ds(r0, _RPT)])
    plsc.subcore_barrier()

    @pl.when(c == 0)
    def _():
        def body(g, carry):
            row = base_row + g * _GR
            pltpu.sync_copy(dst2.at[pl.ds(row, _GR)], idxb)
            pltpu.sync_copy(y0.at[pl.ds(row * _L, _GR * _L)], yb)
            copies = [
                pltpu.async_copy(yb.at[pl.ds(k * _L, _L)],
                                 acc.at[idxb.at[k]], sem, add=True)
                for k in range(_GR)
            ]
            for cp in copies:
                cp.wait()
            return carry

        lax.fori_loop(0, _SNIT, body, 0)

        @pl.when(s < _SEXTRA)
        def _():
            xrow = _SRPT * _NS + s
            pltpu.sync_copy(dst2.at[pl.ds(xrow, 1)], idxb.at[pl.ds(0, 1)])
            pltpu.sync_copy(y0.at[pl.ds(xrow * _L, _L)], yb.at[pl.ds(0, _L)])
            pltpu.async_copy(yb.at[pl.ds(0, _L)],
                             acc.at[idxb.at[0]], sem, add=True).wait()

    @pl.when(c == 1)
    def _():
        def body(g, carry):
            row = base_row + g * _GR
            pltpu.sync_copy(dst2.at[pl.ds(row, _GR)], idxb)
            pltpu.sync_copy(y1.at[pl.ds(row * _L, _GR * _L)], yb)
            copies = [
                pltpu.async_copy(yb.at[pl.ds(k * _L, _L)],
                                 acc.at[idxb.at[k]], sem, add=True)
                for k in range(_GR)
            ]
            for cp in copies:
                cp.wait()
            return carry

        lax.fori_loop(0, _SNIT, body, 0)

        @pl.when(s < _SEXTRA)
        def _():
            xrow = _SRPT * _NS + s
            pltpu.sync_copy(dst2.at[pl.ds(xrow, 1)], idxb.at[pl.ds(0, 1)])
            pltpu.sync_copy(y1.at[pl.ds(xrow * _L, _L)], yb.at[pl.ds(0, _L)])
            pltpu.async_copy(yb.at[pl.ds(0, _L)],
                             acc.at[idxb.at[0]], sem, add=True).wait()

    plsc.subcore_barrier()

    @pl.when(c == 0)
    def _():
        pltpu.sync_copy(acc.at[pl.ds(r0, _RPT)], o0.at[pl.ds(r0, _RPT)])

    @pl.when(c == 1)
    def _():
        pltpu.sync_copy(acc.at[pl.ds(r0, _RPT)], o1.at[pl.ds(r0, _RPT)])


_BE = 6400
_INV_SQRT3 = 1.0 / math.sqrt(3.0)


def _bf(x):
    return x.astype(jnp.bfloat16)


def _tc_body(at, st, w1t, w2t, w3t, wst, wvt, y0, y1):
    inv = at[0:16, :]
    h = jnp.maximum(
        jnp.dot(_bf(w1t[:]), _bf(inv), preferred_element_type=jnp.float32) * 0.25,
        0.0)
    h = jnp.maximum(
        jnp.dot(_bf(w2t[:]), _bf(h), preferred_element_type=jnp.float32) * 0.125,
        0.0)
    f = jnp.dot(_bf(w3t[:]), _bf(h), preferred_element_type=jnp.float32) * 0.125
    fs = f[0:24, :]
    fv = f[24:48, :]
    ex = at[16:17, :]
    ey = at[17:18, :]
    ez = at[18:19, :]
    cw = at[19:20, :]
    s_ = st[0:16, :]
    vx = st[16:24, :]
    vy = st[24:32, :]
    vz = st[32:40, :]
    tp0 = (vx * ex + vy * ey + vz * ez) * _INV_SQRT3
    ms = jnp.concatenate([tp0, s_], axis=0) * fs * cw
    ys = jnp.dot(_bf(wst[:]), _bf(ms), preferred_element_type=jnp.float32)
    mx = jnp.concatenate([s_ * ex, vx], axis=0) * fv * cw
    my = jnp.concatenate([s_ * ey, vy], axis=0) * fv * cw
    mz = jnp.concatenate([s_ * ez, vz], axis=0) * fv * cw
    yx = jnp.dot(_bf(wvt[:]), _bf(mx), preferred_element_type=jnp.float32)
    yy = jnp.dot(_bf(wvt[:]), _bf(my), preferred_element_type=jnp.float32)
    yz = jnp.dot(_bf(wvt[:]), _bf(mz), preferred_element_type=jnp.float32)
    y0[...] = jnp.concatenate([ys, yx], axis=0)
    y1[...] = jnp.concatenate(
        [yy, yz, jnp.zeros((8, yy.shape[1]), jnp.float32)], axis=0)


_tc_compute = pl.pallas_call(
    _tc_body,
    grid=(_E // _BE,),
    in_specs=[
        pl.BlockSpec((24, _BE), lambda i: (0, i)),
        pl.BlockSpec((_D, _BE), lambda i: (0, i)),
        pl.BlockSpec((64, 16), lambda i: (0, 0)),
        pl.BlockSpec((64, 64), lambda i: (0, 0)),
        pl.BlockSpec((48, 64), lambda i: (0, 0)),
        pl.BlockSpec((16, 24), lambda i: (0, 0)),
        pl.BlockSpec((8, 24), lambda i: (0, 0)),
    ],
    out_specs=[
        pl.BlockSpec((_D0, _BE), lambda i: (0, i)),
        pl.BlockSpec((_D0, _BE), lambda i: (0, i)),
    ],
    out_shape=[
        jax.ShapeDtypeStruct((_D0, _E), jnp.float32),
        jax.ShapeDtypeStruct((_D0, _E), jnp.float32),
    ],
)


def kernel(edge_src, edge_dst, edge_weight_cutoff, edge_attr, node_feat,
           W1, W2, W3, Ws, Wv):
    s = node_feat[:, :16]
    v = node_feat[:, 16:].reshape(_N, 8, 3)
    node_tab = jnp.concatenate(
        [s, v[:, :, 0], v[:, :, 1], v[:, :, 2],
         jnp.zeros((_N, 8), jnp.float32)], axis=1)
    src_idx = edge_src.astype(jnp.int32).reshape(_EROWS, _L)
    dst_idx = edge_dst.astype(jnp.int32).reshape(_EROWS, _L)

    sc_gather, sc_scatter = _make_sc_kernels()
    src_feat = sc_gather(node_tab, src_idx)
    attr_all = jnp.concatenate(
        [edge_attr, edge_weight_cutoff[:, None],
         jnp.zeros((_E, 4), jnp.float32)], axis=1)
    y0t, y1t = _tc_compute(attr_all.T, src_feat.T,
                           W1.T, W2.T, W3.T, Ws.T, Wv.T)
    y0 = y0t.T
    y1 = y1t.T
    z = jnp.zeros((_NPAD, _D0), jnp.float32)
    o0, o1 = sc_scatter(y0, y1, dst_idx, z)

    ys = o0[:_N, :16]
    yv = jnp.stack([o0[:_N, 16:24], o1[:_N, :8], o1[:_N, 8:16]],
                   axis=-1).reshape(_N, 24)
    return jnp.concatenate([ys, yv], axis=1)


# speedup vs baseline: 49.0575x; 1.2576x over previous
import functools
import math

import jax
import jax.numpy as jnp
from jax import lax
from jax.experimental import pallas as pl
from jax.experimental.pallas import tpu as pltpu
from jax.experimental.pallas import tpu_sc as plsc

_N = 50000
_E = 800000
_D = 48
_D0 = 24
_D1 = 16
_NC = 2
_NS = 16
_NW = _NC * _NS
_NPAD = 50176
_RPT = _NPAD // _NS

_L = 128
_EROWS = _E // _L
_GRPT = _EROWS // _NW
_GEXTRA = _EROWS - _GRPT * _NW
_GR = 5
_GNIT = _GRPT // _GR
_SGR = 8
_SRPT = 392
_SNIT = 49
_SNIT_LAST = 46


@functools.cache
def _make_sc_kernels():
    mesh = plsc.VectorSubcoreMesh(
        core_axis_name="c", subcore_axis_name="s",
        num_cores=_NC, num_subcores=_NS,
    )
    params = pltpu.CompilerParams(use_tc_tiling_on_sc=False)
    gather = functools.partial(
        pl.kernel,
        out_type=jax.ShapeDtypeStruct((_E, _D), jnp.float32),
        mesh=mesh,
        compiler_params=params,
        scratch_types=[
            pltpu.VMEM((_GRPT + 1, _L), jnp.int32),
            pltpu.VMEM((_GR * _L, _D), jnp.float32),
            pltpu.SemaphoreType.DMA,
        ],
    )(_sc_gather_body)
    scatter = functools.partial(
        pl.kernel,
        out_type=(
            jax.ShapeDtypeStruct((_NPAD, _D0), jnp.float32),
            jax.ShapeDtypeStruct((_NPAD, _D0), jnp.float32),
        ),
        mesh=mesh,
        compiler_params=params,
        scratch_types=[
            pltpu.VMEM((_SGR, _L), jnp.int32),
            pltpu.VMEM((_SGR * _L, _D0), jnp.float32),
            pltpu.VMEM_SHARED((_NPAD, _D0), jnp.float32),
            pltpu.SemaphoreType.DMA,
        ],
    )(_sc_scatter_body)
    return gather, scatter


def _sc_gather_body(tab, idx2, out, idxb, rowb, sem):
    wid = lax.axis_index("s") * _NC + lax.axis_index("c")
    base_row = wid * _GRPT
    pltpu.sync_copy(idx2.at[pl.ds(base_row, _GRPT)], idxb.at[pl.ds(0, _GRPT)])

    @pl.when(wid < _GEXTRA)
    def _():
        pltpu.sync_copy(idx2.at[pl.ds(_GRPT * _NW + wid, 1)],
                        idxb.at[pl.ds(_GRPT, 1)])

    def body(g, carry):
        row = g * _GR
        copies = [
            pltpu.async_copy(tab.at[idxb.at[row + k]],
                             rowb.at[pl.ds(k * _L, _L)], sem)
            for k in range(_GR)
        ]
        for cp in copies:
            cp.wait()
        pltpu.sync_copy(rowb, out.at[pl.ds((base_row + row) * _L, _GR * _L)])
        return carry

    lax.fori_loop(0, _GNIT, body, 0)

    @pl.when(wid < _GEXTRA)
    def _():
        pltpu.async_copy(tab.at[idxb.at[_GRPT]],
                         rowb.at[pl.ds(0, _L)], sem).wait()
        pltpu.sync_copy(rowb.at[pl.ds(0, _L)],
                        out.at[pl.ds((_GRPT * _NW + wid) * _L, _L)])


def _sc_scatter_body(y, dst2, z, o0, o1, idxb, yb, acc, sem):
    c = lax.axis_index("c")
    s = lax.axis_index("s")
    r0 = s * _RPT
    base_row = s * _SRPT
    nblk = jnp.where(s == _NS - 1, _SNIT_LAST, _SNIT)

    pltpu.sync_copy(z.at[pl.ds(r0, _RPT)], acc.at[pl.ds(r0, _RPT)])
    plsc.subcore_barrier()

    def _scatter_from(cbase):
        def body(g, carry):
            row = base_row + g * _SGR
            pltpu.sync_copy(dst2.at[pl.ds(row, _SGR)], idxb)
            pltpu.sync_copy(y.at[pl.ds(row * _L, _SGR * _L), pl.ds(cbase, _D0)],
                            yb)
            copies = [
                pltpu.async_copy(yb.at[pl.ds(k * _L, _L)],
                                 acc.at[idxb.at[k]], sem, add=True)
                for k in range(_SGR)
            ]
            for cp in copies:
                cp.wait()
            return carry

        lax.fori_loop(0, nblk, body, 0)

        @pl.when(s == _NS - 1)
        def _():
            row = _EROWS - 2
            pltpu.sync_copy(dst2.at[pl.ds(row, 2)], idxb.at[pl.ds(0, 2)])
            pltpu.sync_copy(y.at[pl.ds(row * _L, 2 * _L), pl.ds(cbase, _D0)],
                            yb.at[pl.ds(0, 2 * _L)])
            copies = [
                pltpu.async_copy(yb.at[pl.ds(k * _L, _L)],
                                 acc.at[idxb.at[k]], sem, add=True)
                for k in range(2)
            ]
            for cp in copies:
                cp.wait()

    @pl.when(c == 0)
    def _():
        _scatter_from(0)

    @pl.when(c == 1)
    def _():
        _scatter_from(_D0)

    plsc.subcore_barrier()

    @pl.when(c == 0)
    def _():
        pltpu.sync_copy(acc.at[pl.ds(r0, _RPT)], o0.at[pl.ds(r0, _RPT)])

    @pl.when(c == 1)
    def _():
        pltpu.sync_copy(acc.at[pl.ds(r0, _RPT)], o1.at[pl.ds(r0, _RPT)])


_BE = 6400
_INV_SQRT3 = 1.0 / math.sqrt(3.0)


def _bf(x):
    return x.astype(jnp.bfloat16)


def _tc_body(attr, srcb, i24, i48, w1t, w2t, w3t, ws, wv, y01):
    at = lax.dot_general(i24[:], attr[...], (((1,), (1,)), ((), ())),
                         preferred_element_type=jnp.float32)
    st = lax.dot_general(i48[:], srcb[...], (((1,), (1,)), ((), ())),
                         preferred_element_type=jnp.float32)
    inv = at[0:16, :]
    h = jnp.maximum(
        jnp.dot(_bf(w1t[:]), _bf(inv), preferred_element_type=jnp.float32) * 0.25,
        0.0)
    h = jnp.maximum(
        jnp.dot(_bf(w2t[:]), _bf(h), preferred_element_type=jnp.float32) * 0.125,
        0.0)
    f = jnp.dot(_bf(w3t[:]), _bf(h), preferred_element_type=jnp.float32) * 0.125
    fs = f[0:24, :]
    fv = f[24:48, :]
    ex = at[16:17, :]
    ey = at[17:18, :]
    ez = at[18:19, :]
    cw = at[19:20, :]
    s_ = st[0:16, :]
    vx = st[16:24, :]
    vy = st[24:32, :]
    vz = st[32:40, :]
    tp0 = (vx * ex + vy * ey + vz * ez) * _INV_SQRT3
    ms = jnp.concatenate([tp0, s_], axis=0) * fs * cw
    mx = jnp.concatenate([s_ * ex, vx], axis=0) * fv * cw
    my = jnp.concatenate([s_ * ey, vy], axis=0) * fv * cw
    mz = jnp.concatenate([s_ * ez, vz], axis=0) * fv * cw
    contract0 = (((0,), (0,)), ((), ()))
    ys = lax.dot_general(_bf(ms), _bf(ws[:]), contract0,
                         preferred_element_type=jnp.float32)
    yx = lax.dot_general(_bf(mx), _bf(wv[:]), contract0,
                         preferred_element_type=jnp.float32)
    yy = lax.dot_general(_bf(my), _bf(wv[:]), contract0,
                         preferred_element_type=jnp.float32)
    yz = lax.dot_general(_bf(mz), _bf(wv[:]), contract0,
                         preferred_element_type=jnp.float32)
    y01[...] = jnp.concatenate(
        [ys, yx, yy, yz, jnp.zeros((ys.shape[0], 88), jnp.float32)], axis=1)


_tc_compute = pl.pallas_call(
    _tc_body,
    grid=(_E // _BE,),
    in_specs=[
        pl.BlockSpec((_BE, 24), lambda i: (i, 0)),
        pl.BlockSpec((_BE, _D), lambda i: (i, 0)),
        pl.BlockSpec((24, 24), lambda i: (0, 0)),
        pl.BlockSpec((_D, _D), lambda i: (0, 0)),
        pl.BlockSpec((64, 16), lambda i: (0, 0)),
        pl.BlockSpec((64, 64), lambda i: (0, 0)),
        pl.BlockSpec((48, 64), lambda i: (0, 0)),
        pl.BlockSpec((24, 16), lambda i: (0, 0)),
        pl.BlockSpec((24, 8), lambda i: (0, 0)),
    ],
    out_specs=pl.BlockSpec((_BE, 128), lambda i: (i, 0)),
    out_shape=jax.ShapeDtypeStruct((_E, 128), jnp.float32),
)


def kernel(edge_src, edge_dst, edge_weight_cutoff, edge_attr, node_feat,
           W1, W2, W3, Ws, Wv):
    s = node_feat[:, :16]
    v = node_feat[:, 16:].reshape(_N, 8, 3)
    node_tab = jnp.concatenate(
        [s, v[:, :, 0], v[:, :, 1], v[:, :, 2],
         jnp.zeros((_N, 8), jnp.float32)], axis=1)
    src_idx = edge_src.astype(jnp.int32).reshape(_EROWS, _L)
    dst_idx = edge_dst.astype(jnp.int32).reshape(_EROWS, _L)

    sc_gather, sc_scatter = _make_sc_kernels()
    src_feat = sc_gather(node_tab, src_idx)
    attr_all = jnp.concatenate(
        [edge_attr, edge_weight_cutoff[:, None],
         jnp.zeros((_E, 4), jnp.float32)], axis=1)
    y = _tc_compute(attr_all, src_feat,
                    jnp.eye(24, dtype=jnp.float32),
                    jnp.eye(_D, dtype=jnp.float32),
                    W1.T, W2.T, W3.T, Ws, Wv)
    z = jnp.zeros((_NPAD, _D0), jnp.float32)
    o0, o1 = sc_scatter(y, dst_idx, z)

    ys = o0[:_N, :16]
    yv = jnp.stack([o0[:_N, 16:24], o1[:_N, :8], o1[:_N, 8:16]],
                   axis=-1).reshape(_N, 24)
    return jnp.concatenate([ys, yv], axis=1)


# speedup vs baseline: 61.0850x; 1.2452x over previous
import functools
import math

import jax
import jax.numpy as jnp
from jax import lax
from jax.experimental import pallas as pl
from jax.experimental.pallas import tpu as pltpu
from jax.experimental.pallas import tpu_sc as plsc

_N = 50000
_E = 800000
_D = 48
_DW = 128
_D0 = 24
_D1 = 16
_NC = 2
_NS = 16
_NW = _NC * _NS
_NPAD = 50176
_RPT = _NPAD // _NS

_L = 128
_EROWS = _E // _L
_GRPT = _EROWS // _NW
_GEXTRA = _EROWS - _GRPT * _NW
_GR = 5
_GNIT = _GRPT // _GR
_SGR = 8
_SRPT = 392
_SNIT = 49
_SNIT_LAST = 46


@functools.cache
def _make_sc_kernels():
    mesh = plsc.VectorSubcoreMesh(
        core_axis_name="c", subcore_axis_name="s",
        num_cores=_NC, num_subcores=_NS,
    )
    params = pltpu.CompilerParams(use_tc_tiling_on_sc=False)
    gather = functools.partial(
        pl.kernel,
        out_type=jax.ShapeDtypeStruct((_E, _DW), jnp.float32),
        mesh=mesh,
        compiler_params=params,
        scratch_types=[
            pltpu.VMEM((_GRPT + 1, _L), jnp.int32),
            pltpu.VMEM((_GR * _L, _DW), jnp.float32),
            pltpu.SemaphoreType.DMA,
        ],
    )(_sc_gather_body)
    scatter = functools.partial(
        pl.kernel,
        out_type=(
            jax.ShapeDtypeStruct((_NPAD, _D0), jnp.float32),
            jax.ShapeDtypeStruct((_NPAD, _D0), jnp.float32),
        ),
        mesh=mesh,
        compiler_params=params,
        scratch_types=[
            pltpu.VMEM((_SGR, _L), jnp.int32),
            pltpu.VMEM((_SGR * _L, _D0), jnp.float32),
            pltpu.VMEM_SHARED((_NPAD, _D0), jnp.float32),
            pltpu.SemaphoreType.DMA,
        ],
    )(_sc_scatter_body)
    return gather, scatter


def _sc_gather_body(tab, idx2, out, idxb, rowb, sem):
    wid = lax.axis_index("s") * _NC + lax.axis_index("c")
    base_row = wid * _GRPT
    pltpu.sync_copy(idx2.at[pl.ds(base_row, _GRPT)], idxb.at[pl.ds(0, _GRPT)])

    @pl.when(wid < _GEXTRA)
    def _():
        pltpu.sync_copy(idx2.at[pl.ds(_GRPT * _NW + wid, 1)],
                        idxb.at[pl.ds(_GRPT, 1)])

    def body(g, carry):
        row = g * _GR
        copies = [
            pltpu.async_copy(tab.at[idxb.at[row + k]],
                             rowb.at[pl.ds(k * _L, _L)], sem)
            for k in range(_GR)
        ]
        for cp in copies:
            cp.wait()
        pltpu.sync_copy(rowb, out.at[pl.ds((base_row + row) * _L, _GR * _L)])
        return carry

    lax.fori_loop(0, _GNIT, body, 0)

    @pl.when(wid < _GEXTRA)
    def _():
        pltpu.async_copy(tab.at[idxb.at[_GRPT]],
                         rowb.at[pl.ds(0, _L)], sem).wait()
        pltpu.sync_copy(rowb.at[pl.ds(0, _L)],
                        out.at[pl.ds((_GRPT * _NW + wid) * _L, _L)])


def _sc_scatter_body(y, dst2, z, o0, o1, idxb, yb, acc, sem):
    c = lax.axis_index("c")
    s = lax.axis_index("s")
    r0 = s * _RPT
    base_row = s * _SRPT
    nblk = jnp.where(s == _NS - 1, _SNIT_LAST, _SNIT)

    pltpu.sync_copy(z.at[pl.ds(r0, _RPT)], acc.at[pl.ds(r0, _RPT)])
    plsc.subcore_barrier()

    def _scatter_from(cbase):
        def body(g, carry):
            row = base_row + g * _SGR
            pltpu.sync_copy(dst2.at[pl.ds(row, _SGR)], idxb)
            pltpu.sync_copy(y.at[pl.ds(row * _L, _SGR * _L), pl.ds(cbase, _D0)],
                            yb)
            copies = [
                pltpu.async_copy(yb.at[pl.ds(k * _L, _L)],
                                 acc.at[idxb.at[k]], sem, add=True)
                for k in range(_SGR)
            ]
            for cp in copies:
                cp.wait()
            return carry

        lax.fori_loop(0, nblk, body, 0)

        @pl.when(s == _NS - 1)
        def _():
            row = _EROWS - 2
            pltpu.sync_copy(dst2.at[pl.ds(row, 2)], idxb.at[pl.ds(0, 2)])
            pltpu.sync_copy(y.at[pl.ds(row * _L, 2 * _L), pl.ds(cbase, _D0)],
                            yb.at[pl.ds(0, 2 * _L)])
            copies = [
                pltpu.async_copy(yb.at[pl.ds(k * _L, _L)],
                                 acc.at[idxb.at[k]], sem, add=True)
                for k in range(2)
            ]
            for cp in copies:
                cp.wait()

    @pl.when(c == 0)
    def _():
        _scatter_from(0)

    @pl.when(c == 1)
    def _():
        _scatter_from(_D0)

    plsc.subcore_barrier()

    @pl.when(c == 0)
    def _():
        pltpu.sync_copy(acc.at[pl.ds(r0, _RPT)], o0.at[pl.ds(r0, _RPT)])

    @pl.when(c == 1)
    def _():
        pltpu.sync_copy(acc.at[pl.ds(r0, _RPT)], o1.at[pl.ds(r0, _RPT)])


_BE = 6400
_INV_SQRT3 = 1.0 / math.sqrt(3.0)


def _bf(x):
    return x.astype(jnp.bfloat16)


def _tc_body(attr, srcb, i24, sel48, sel40, w1t, w2t, w3t, wst, wvt, y01):
    at = lax.dot_general(_bf(i24[:]), _bf(attr[...]), (((1,), (1,)), ((), ())),
                         preferred_element_type=jnp.float32)
    st = lax.dot_general(_bf(sel48[:]), _bf(srcb[...]), (((1,), (1,)), ((), ())),
                         preferred_element_type=jnp.float32)
    inv = at[0:16, :]
    h = jnp.maximum(
        jnp.dot(_bf(w1t[:]), _bf(inv), preferred_element_type=jnp.float32) * 0.25,
        0.0)
    h = jnp.maximum(
        jnp.dot(_bf(w2t[:]), _bf(h), preferred_element_type=jnp.float32) * 0.125,
        0.0)
    f = jnp.dot(_bf(w3t[:]), _bf(h), preferred_element_type=jnp.float32) * 0.125
    fs = f[0:24, :]
    fv = f[24:48, :]
    ex = at[16:17, :]
    ey = at[17:18, :]
    ez = at[18:19, :]
    cw = at[19:20, :]
    s_ = st[0:16, :]
    vx = st[16:24, :]
    vy = st[24:32, :]
    vz = st[32:40, :]
    tp0 = (vx * ex + vy * ey + vz * ez) * _INV_SQRT3
    ms = jnp.concatenate([tp0, s_], axis=0) * fs * cw
    mx = jnp.concatenate([s_ * ex, vx], axis=0) * fv * cw
    my = jnp.concatenate([s_ * ey, vy], axis=0) * fv * cw
    mz = jnp.concatenate([s_ * ez, vz], axis=0) * fv * cw
    ys = jnp.dot(_bf(wst[:]), _bf(ms), preferred_element_type=jnp.float32)
    yx = jnp.dot(_bf(wvt[:]), _bf(mx), preferred_element_type=jnp.float32)
    yy = jnp.dot(_bf(wvt[:]), _bf(my), preferred_element_type=jnp.float32)
    yz = jnp.dot(_bf(wvt[:]), _bf(mz), preferred_element_type=jnp.float32)
    yt = jnp.concatenate([ys, yx, yy, yz], axis=0)
    y01[...] = lax.dot_general(yt, sel40[:], (((0,), (0,)), ((), ())),
                               preferred_element_type=jnp.float32)


_tc_compute = pl.pallas_call(
    _tc_body,
    grid=(_E // _BE,),
    in_specs=[
        pl.BlockSpec((_BE, 24), lambda i: (i, 0)),
        pl.BlockSpec((_BE, _DW), lambda i: (i, 0)),
        pl.BlockSpec((24, 24), lambda i: (0, 0)),
        pl.BlockSpec((_D, _DW), lambda i: (0, 0)),
        pl.BlockSpec((40, _DW), lambda i: (0, 0)),
        pl.BlockSpec((64, 16), lambda i: (0, 0)),
        pl.BlockSpec((64, 64), lambda i: (0, 0)),
        pl.BlockSpec((48, 64), lambda i: (0, 0)),
        pl.BlockSpec((16, 24), lambda i: (0, 0)),
        pl.BlockSpec((8, 24), lambda i: (0, 0)),
    ],
    out_specs=pl.BlockSpec((_BE, 128), lambda i: (i, 0)),
    out_shape=jax.ShapeDtypeStruct((_E, 128), jnp.float32),
    compiler_params=pltpu.CompilerParams(fuse_transposed_lhs_in_matmul=True),
)


def kernel(edge_src, edge_dst, edge_weight_cutoff, edge_attr, node_feat,
           W1, W2, W3, Ws, Wv):
    s = node_feat[:, :16]
    v = node_feat[:, 16:].reshape(_N, 8, 3)
    node_tab = jnp.concatenate(
        [s, v[:, :, 0], v[:, :, 1], v[:, :, 2],
         jnp.zeros((_N, _DW - 40), jnp.float32)], axis=1)
    src_idx = edge_src.astype(jnp.int32).reshape(_EROWS, _L)
    dst_idx = edge_dst.astype(jnp.int32).reshape(_EROWS, _L)

    sc_gather, sc_scatter = _make_sc_kernels()
    src_feat = sc_gather(node_tab, src_idx)
    attr_all = jnp.concatenate(
        [edge_attr, edge_weight_cutoff[:, None],
         jnp.zeros((_E, 4), jnp.float32)], axis=1)
    y = _tc_compute(attr_all, src_feat,
                    jnp.eye(24, dtype=jnp.float32),
                    jnp.eye(_D, _DW, dtype=jnp.float32),
                    jnp.eye(40, _DW, dtype=jnp.float32),
                    W1.T, W2.T, W3.T, Ws.T, Wv.T)
    z = jnp.zeros((_NPAD, _D0), jnp.float32)
    o0, o1 = sc_scatter(y, dst_idx, z)

    ys = o0[:_N, :16]
    yv = jnp.stack([o0[:_N, 16:24], o1[:_N, :8], o1[:_N, 8:16]],
                   axis=-1).reshape(_N, 24)
    return jnp.concatenate([ys, yv], axis=1)


# speedup vs baseline: 76.2591x; 1.2484x over previous
import functools
import math

import numpy as np

import jax
import jax.numpy as jnp
from jax import lax
from jax.experimental import pallas as pl
from jax.experimental.pallas import tpu as pltpu
from jax.experimental.pallas import tpu_sc as plsc

_N = 50000
_E = 800000
_D = 48
_DW = 128
_D0 = 24
_D1 = 16
_NC = 2
_NS = 16
_NW = _NC * _NS
_NPAD = 50176
_RPT = _NPAD // _NS

_L = 128
_EROWS = _E // _L
_GRPT = _EROWS // _NW
_GEXTRA = _EROWS - _GRPT * _NW
_GR = 5
_GNIT = _GRPT // _GR
_SGR = 8
_SRPT = 392
_SNIT = 49
_SNIT_LAST = 46


@functools.cache
def _make_sc_kernels():
    mesh = plsc.VectorSubcoreMesh(
        core_axis_name="c", subcore_axis_name="s",
        num_cores=_NC, num_subcores=_NS,
    )
    params = pltpu.CompilerParams(use_tc_tiling_on_sc=False)
    gather = functools.partial(
        pl.kernel,
        out_type=jax.ShapeDtypeStruct((_E, _DW), jnp.float32),
        mesh=mesh,
        compiler_params=params,
        scratch_types=[
            pltpu.VMEM((_GRPT + 1, _L), jnp.int32),
            pltpu.VMEM((_GR * _L, _DW), jnp.float32),
            pltpu.SemaphoreType.DMA,
        ],
    )(_sc_gather_body)
    scatter = functools.partial(
        pl.kernel,
        out_type=(
            jax.ShapeDtypeStruct((_NPAD, _D0), jnp.float32),
            jax.ShapeDtypeStruct((_NPAD, _D0), jnp.float32),
        ),
        mesh=mesh,
        compiler_params=params,
        scratch_types=[
            pltpu.VMEM((_SGR, _L), jnp.int32),
            pltpu.VMEM((_SGR * _L, _D0), jnp.float32),
            pltpu.VMEM_SHARED((_NPAD, _D0), jnp.float32),
            pltpu.SemaphoreType.DMA,
        ],
    )(_sc_scatter_body)
    return gather, scatter


def _sc_gather_body(tab, idx2, out, idxb, rowb, sem):
    wid = lax.axis_index("s") * _NC + lax.axis_index("c")
    base_row = wid * _GRPT
    pltpu.sync_copy(idx2.at[pl.ds(base_row, _GRPT)], idxb.at[pl.ds(0, _GRPT)])

    @pl.when(wid < _GEXTRA)
    def _():
        pltpu.sync_copy(idx2.at[pl.ds(_GRPT * _NW + wid, 1)],
                        idxb.at[pl.ds(_GRPT, 1)])

    def body(g, carry):
        row = g * _GR
        copies = [
            pltpu.async_copy(tab.at[idxb.at[row + k]],
                             rowb.at[pl.ds(k * _L, _L)], sem)
            for k in range(_GR)
        ]
        for cp in copies:
            cp.wait()
        pltpu.sync_copy(rowb.at[pl.ds(0, _GR * _L), pl.ds(0, _D)],
                        out.at[pl.ds((base_row + row) * _L, _GR * _L),
                               pl.ds(0, _D)])
        return carry

    lax.fori_loop(0, _GNIT, body, 0)

    @pl.when(wid < _GEXTRA)
    def _():
        pltpu.async_copy(tab.at[idxb.at[_GRPT]],
                         rowb.at[pl.ds(0, _L)], sem).wait()
        pltpu.sync_copy(rowb.at[pl.ds(0, _L), pl.ds(0, _D)],
                        out.at[pl.ds((_GRPT * _NW + wid) * _L, _L),
                               pl.ds(0, _D)])


def _sc_scatter_body(y, dst2, z, o0, o1, idxb, yb, acc, sem):
    c = lax.axis_index("c")
    s = lax.axis_index("s")
    r0 = s * _RPT
    base_row = s * _SRPT
    nblk = jnp.where(s == _NS - 1, _SNIT_LAST, _SNIT)

    pltpu.sync_copy(z.at[pl.ds(r0, _RPT)], acc.at[pl.ds(r0, _RPT)])
    plsc.subcore_barrier()

    def _scatter_from(cbase):
        def body(g, carry):
            row = base_row + g * _SGR
            pltpu.sync_copy(dst2.at[pl.ds(row, _SGR)], idxb)
            pltpu.sync_copy(y.at[pl.ds(row * _L, _SGR * _L), pl.ds(cbase, _D0)],
                            yb)
            copies = [
                pltpu.async_copy(yb.at[pl.ds(k * _L, _L)],
                                 acc.at[idxb.at[k]], sem, add=True)
                for k in range(_SGR)
            ]
            for cp in copies:
                cp.wait()
            return carry

        lax.fori_loop(0, nblk, body, 0)

        @pl.when(s == _NS - 1)
        def _():
            row = _EROWS - 2
            pltpu.sync_copy(dst2.at[pl.ds(row, 2)], idxb.at[pl.ds(0, 2)])
            pltpu.sync_copy(y.at[pl.ds(row * _L, 2 * _L), pl.ds(cbase, _D0)],
                            yb.at[pl.ds(0, 2 * _L)])
            copies = [
                pltpu.async_copy(yb.at[pl.ds(k * _L, _L)],
                                 acc.at[idxb.at[k]], sem, add=True)
                for k in range(2)
            ]
            for cp in copies:
                cp.wait()

    @pl.when(c == 0)
    def _():
        _scatter_from(0)

    @pl.when(c == 1)
    def _():
        _scatter_from(_D0)

    plsc.subcore_barrier()

    @pl.when(c == 0)
    def _():
        pltpu.sync_copy(acc.at[pl.ds(r0, _RPT)], o0.at[pl.ds(r0, _RPT)])

    @pl.when(c == 1)
    def _():
        pltpu.sync_copy(acc.at[pl.ds(r0, _RPT)], o1.at[pl.ds(r0, _RPT)])


_BE = 6400
_INV_SQRT3 = 1.0 / math.sqrt(3.0)


def _bf(x):
    return x.astype(jnp.bfloat16)


def _selperm():
    cols = np.concatenate([np.arange(16), 16 + 3 * np.arange(8),
                           17 + 3 * np.arange(8), 18 + 3 * np.arange(8)])
    sel = np.zeros((40, _DW), np.float32)
    sel[np.arange(40), cols] = 1.0
    return jnp.asarray(sel)


def _tc_body(attr, srcb, i24, selp, w1t, w2t, w3t, wst, wvt, y01):
    at = lax.dot_general(_bf(i24[:]), _bf(attr[...]), (((1,), (1,)), ((), ())),
                         preferred_element_type=jnp.float32)
    st = lax.dot_general(_bf(selp[:, 0:40]), _bf(srcb[...][:, 0:40]),
                         (((1,), (1,)), ((), ())),
                         preferred_element_type=jnp.float32)
    inv = at[0:16, :]
    h = jnp.maximum(
        jnp.dot(_bf(w1t[:]), _bf(inv), preferred_element_type=jnp.float32) * 0.25,
        0.0)
    h = jnp.maximum(
        jnp.dot(_bf(w2t[:]), _bf(h), preferred_element_type=jnp.float32) * 0.125,
        0.0)
    f = jnp.dot(_bf(w3t[:]), _bf(h), preferred_element_type=jnp.float32) * 0.125
    fs = f[0:24, :]
    fv = f[24:48, :]
    ex = at[16:17, :]
    ey = at[17:18, :]
    ez = at[18:19, :]
    cw = at[19:20, :]
    s_ = st[0:16, :]
    vx = st[16:24, :]
    vy = st[24:32, :]
    vz = st[32:40, :]
    tp0 = (vx * ex + vy * ey + vz * ez) * _INV_SQRT3
    ms = jnp.concatenate([tp0, s_], axis=0) * fs * cw
    mx = jnp.concatenate([s_ * ex, vx], axis=0) * fv * cw
    my = jnp.concatenate([s_ * ey, vy], axis=0) * fv * cw
    mz = jnp.concatenate([s_ * ez, vz], axis=0) * fv * cw
    ys = jnp.dot(_bf(wst[:]), _bf(ms), preferred_element_type=jnp.float32)
    yx = jnp.dot(_bf(wvt[:]), _bf(mx), preferred_element_type=jnp.float32)
    yy = jnp.dot(_bf(wvt[:]), _bf(my), preferred_element_type=jnp.float32)
    yz = jnp.dot(_bf(wvt[:]), _bf(mz), preferred_element_type=jnp.float32)
    yt = jnp.concatenate([ys, yx, yy, yz], axis=0)
    y01[...] = lax.dot_general(yt, selp[:], (((0,), (0,)), ((), ())),
                               preferred_element_type=jnp.float32)


_tc_compute = pl.pallas_call(
    _tc_body,
    grid=(_E // _BE,),
    in_specs=[
        pl.BlockSpec((_BE, 24), lambda i: (i, 0)),
        pl.BlockSpec((_BE, _DW), lambda i: (i, 0)),
        pl.BlockSpec((24, 24), lambda i: (0, 0)),
        pl.BlockSpec((40, _DW), lambda i: (0, 0)),
        pl.BlockSpec((64, 16), lambda i: (0, 0)),
        pl.BlockSpec((64, 64), lambda i: (0, 0)),
        pl.BlockSpec((48, 64), lambda i: (0, 0)),
        pl.BlockSpec((16, 24), lambda i: (0, 0)),
        pl.BlockSpec((8, 24), lambda i: (0, 0)),
    ],
    out_specs=pl.BlockSpec((_BE, 128), lambda i: (i, 0)),
    out_shape=jax.ShapeDtypeStruct((_E, 128), jnp.float32),
    compiler_params=pltpu.CompilerParams(fuse_transposed_lhs_in_matmul=True),
)


def kernel(edge_src, edge_dst, edge_weight_cutoff, edge_attr, node_feat,
           W1, W2, W3, Ws, Wv):
    node_tab = jnp.concatenate(
        [node_feat, jnp.zeros((_N, _DW - 40), jnp.float32)], axis=1)
    src_idx = edge_src.astype(jnp.int32).reshape(_EROWS, _L)
    dst_idx = edge_dst.astype(jnp.int32).reshape(_EROWS, _L)

    sc_gather, sc_scatter = _make_sc_kernels()
    src_feat = sc_gather(node_tab, src_idx)
    attr_all = jnp.concatenate(
        [edge_attr, edge_weight_cutoff[:, None],
         jnp.zeros((_E, 4), jnp.float32)], axis=1)
    y = _tc_compute(attr_all, src_feat,
                    jnp.eye(24, dtype=jnp.float32), _selperm(),
                    W1.T, W2.T, W3.T, Ws.T, Wv.T)
    z = jnp.zeros((_NPAD, _D0), jnp.float32)
    o0, o1 = sc_scatter(y, dst_idx, z)

    return jnp.concatenate([o0[:_N, :], o1[:_N, :16]], axis=1)
